# baseline (speedup 1.0000x reference)
import functools

import jax
import jax.numpy as jnp
from jax import lax
from jax.experimental import pallas as pl
from jax.experimental.pallas import tpu as pltpu
from jax.experimental.pallas import tpu_sc as plsc

B = 16384
C = 50
D = 32

NC = 2
NS = 16
NW = NC * NS
BPW = B // NW
CH = 16
NCHUNK = BPW // CH
TAG_SUB = 100
SUBS = CH * C // TAG_SUB
PR = CH * C


def _sc_kernel(user_h, item_h, trow_h, tsub_h, tu_h, ti_h, tt_h, out_h,
               uraw_v, iraw_v, uidx_v, iidx_v, usub_v, isub_v, tidx_v, tsub_v,
               urows_v, irows_v, trows_v, out_v, sem):
    wid = lax.axis_index("s") * NC + lax.axis_index("c")
    iota = lax.iota(jnp.int32, 16)

    @pl.loop(0, NCHUNK)
    def _chunk(ch):
        b0 = pl.multiple_of(wid * BPW + ch * CH, CH)
        trow0 = pl.multiple_of(b0 * C // TAG_SUB, SUBS)

        pltpu.sync_copy(user_h.at[pl.ds(b0, CH)], uraw_v)
        pltpu.sync_copy(item_h.at[pl.ds(b0, CH)], iraw_v)
        pltpu.sync_copy(trow_h.at[pl.ds(trow0, SUBS)], tidx_v)
        pltpu.sync_copy(tsub_h.at[pl.ds(b0 * C, PR)], tsub_v)
        uidx_v[...] = lax.shift_right_logical(uraw_v[...], 2)
        iidx_v[...] = lax.shift_right_logical(iraw_v[...], 2)
        usub_v[...] = (uraw_v[...] & 3) * D
        isub_v[...] = (iraw_v[...] & 3) * D

        copies = [pltpu.async_copy(tu_h.at[uidx_v], urows_v, sem),
                  pltpu.async_copy(ti_h.at[iidx_v], irows_v, sem)]
        for j in range(SUBS):
            copies.append(
                pltpu.async_copy(tt_h.at[tidx_v.at[j]],
                                 trows_v.at[pl.ds(j * TAG_SUB, TAG_SUB)],
                                 sem))
        for cp in copies:
            cp.wait()

        @pl.loop(0, CH)
        def _row(b):
            ub = jnp.full((16,), b, jnp.int32)
            uoff = plsc.load_gather(usub_v, [ub])
            ioff = plsc.load_gather(isub_v, [ub])
            s = [plsc.load_gather(urows_v, [ub, uoff + d + iota])
                 + plsc.load_gather(irows_v, [ub, ioff + d + iota])
                 for d in (0, 16)]
            prow = b * C
            for g in range(4):
                cvec = iota + g * 16
                pvec = jnp.minimum(cvec + prow, PR - 1)
                mask = (cvec < C) if g == 3 else None
                toff = plsc.load_gather(tsub_v, [pvec], mask=mask)
                acc = jnp.zeros((16,), jnp.float32)
                for d in range(D):
                    sd = s[d // 16][d % 16]
                    tv = plsc.load_gather(trows_v, [pvec, toff + d], mask=mask)
                    acc = acc + sd * tv
                plsc.store_scatter(out_v, [cvec + prow], acc, mask=mask)

        pltpu.sync_copy(out_v, out_h.at[pl.ds(b0 * C, PR)])


def kernel(user, item, tag, tag_type, table_u, table_i, table_t):
    del tag_type
    user = user.astype(jnp.int32)
    item = item.astype(jnp.int32)
    tag = tag.astype(jnp.int32)
    trow = lax.shift_right_logical(tag, 2).reshape(B * C // TAG_SUB, TAG_SUB)
    tsub = ((tag & 3) * D).reshape(B * C)

    mesh = plsc.VectorSubcoreMesh(core_axis_name="c", subcore_axis_name="s")
    run = functools.partial(
        pl.kernel,
        out_type=jax.ShapeDtypeStruct((B * C,), jnp.float32),
        mesh=mesh,
        compiler_params=pltpu.CompilerParams(needs_layout_passes=False,
                                             use_tc_tiling_on_sc=False),
        scratch_types=[
            pltpu.VMEM((CH,), jnp.int32),
            pltpu.VMEM((CH,), jnp.int32),
            pltpu.VMEM((CH,), jnp.int32),
            pltpu.VMEM((CH,), jnp.int32),
            pltpu.VMEM((CH,), jnp.int32),
            pltpu.VMEM((CH,), jnp.int32),
            pltpu.VMEM((SUBS, TAG_SUB), jnp.int32),
            pltpu.VMEM((PR,), jnp.int32),
            pltpu.VMEM((CH, 4 * D), jnp.float32),
            pltpu.VMEM((CH, 4 * D), jnp.float32),
            pltpu.VMEM((PR, 4 * D), jnp.float32),
            pltpu.VMEM((PR,), jnp.float32),
            pltpu.SemaphoreType.DMA,
        ],
    )(_sc_kernel)
    return run(user, item, trow, tsub, table_u.reshape(250000, 128),
               table_i.reshape(250000, 128),
               table_t.reshape(25000, 128)).reshape(B, C)
